# Initial kernel scaffold; baseline (speedup 1.0000x reference)
import jax
import jax.numpy as jnp
from jax.experimental import pallas as pl


def kernel(x, edge_index, edge_attr, batch, n_node_neurons, n_node_features, n_edge_neurons, n_edge_features, n_gc_layers, n_s_layers, n_ts_layers, use_molecular_descriptors, global_feats, global_mean, global_std, params):
    raise NotImplementedError("write your pallas kernel here")


# speedup vs baseline: 2.0624x; 2.0624x over previous
import functools

import jax
import jax.numpy as jnp
from jax import lax
from jax.experimental import pallas as pl
from jax.experimental.pallas import tpu as pltpu
from jax.experimental.pallas import tpu_sc as plsc

NC, NS, LANES = 2, 16, 16
KCH = 80


def _gather_body(nchunks, x_hbm, dst_hbm, src_hbm, xd_hbm, xs_hbm,
                 idx_v, rows_v, sem):
    c = lax.axis_index("c")
    s = lax.axis_index("s")
    wid = c * NS + s
    base = wid * (nchunks * KCH)

    def chunk(i, _):
        off = base + i * KCH
        pltpu.sync_copy(dst_hbm.at[pl.ds(off, KCH)], idx_v)
        pltpu.async_copy(x_hbm.at[idx_v], rows_v, sem).wait()
        pltpu.sync_copy(rows_v, xd_hbm.at[pl.ds(off, KCH)])
        pltpu.sync_copy(src_hbm.at[pl.ds(off, KCH)], idx_v)
        pltpu.async_copy(x_hbm.at[idx_v], rows_v, sem).wait()
        pltpu.sync_copy(rows_v, xs_hbm.at[pl.ds(off, KCH)])
        return 0

    lax.fori_loop(0, nchunks, chunk, 0)


def _sc_gather(x, dst, src):
    E = dst.shape[0]
    N, D = x.shape
    nchunks = E // (NC * NS * KCH)
    mesh = plsc.VectorSubcoreMesh(core_axis_name="c", subcore_axis_name="s")
    f = pl.kernel(
        functools.partial(_gather_body, nchunks),
        out_type=[jax.ShapeDtypeStruct((E, D), jnp.float32),
                  jax.ShapeDtypeStruct((E, D), jnp.float32)],
        mesh=mesh,
        scratch_types=[
            pltpu.VMEM((KCH,), jnp.int32),
            pltpu.VMEM((KCH, D), jnp.float32),
            pltpu.SemaphoreType.DMA,
        ],
    )
    return f(x, dst, src)


def _scatter_body(nchunks, npad, msg_hbm, dst_hbm, zeros_hbm, out_hbm,
                  idx_v, rows_v, acc_sh):
    c = lax.axis_index("c")
    s = lax.axis_index("s")
    wid = c * NS + s
    base = wid * (nchunks * KCH)

    @pl.when(s == 0)
    def _zero():
        pltpu.sync_copy(zeros_hbm, acc_sh)
    plsc.subcore_barrier()

    def chunk(i, _):
        off = base + i * KCH
        pltpu.sync_copy(dst_hbm.at[pl.ds(off, KCH)], idx_v)
        pltpu.sync_copy(msg_hbm.at[pl.ds(off, KCH)], rows_v)
        pltpu.sync_copy(rows_v, acc_sh.at[idx_v], add=True)
        return 0

    lax.fori_loop(0, nchunks, chunk, 0)
    plsc.subcore_barrier()

    @pl.when(s == 0)
    def _readback():
        pltpu.sync_copy(acc_sh, out_hbm.at[pl.ds(c * npad, npad)])


def _sc_scatter(msg, dst, zeros, N):
    E, D = msg.shape
    nchunks = E // (NC * NS * KCH)
    npad = zeros.shape[0]
    mesh = plsc.VectorSubcoreMesh(core_axis_name="c", subcore_axis_name="s")
    f = pl.kernel(
        functools.partial(_scatter_body, nchunks, npad),
        out_type=jax.ShapeDtypeStruct((NC * npad, D), jnp.float32),
        mesh=mesh,
        scratch_types=[
            pltpu.VMEM((KCH,), jnp.int32),
            pltpu.VMEM((KCH, D), jnp.float32),
            pltpu.VMEM_SHARED((npad, D), jnp.float32),
        ],
    )
    return f(msg, dst, zeros)


def _sigmoid(z):
    return 1.0 / (1.0 + jnp.exp(-z))


def _softplus(z):
    return jnp.maximum(z, 0.0) + jnp.log(1.0 + jnp.exp(-jnp.abs(z)))


def _msg_body(D, DE, xd_ref, xs_ref, ea_ref, wf_ref, ws_ref, bf_ref, bs_ref,
              o_ref):
    xd = xd_ref[...]
    xs = xs_ref[...]
    ea = ea_ref[...]
    wf = wf_ref[...]
    ws = ws_ref[...]
    f32 = jnp.float32
    zf = (jnp.dot(xd, wf[0:D], preferred_element_type=f32)
          + jnp.dot(xs, wf[D:2 * D], preferred_element_type=f32)
          + jnp.dot(ea, wf[2 * D:2 * D + DE], preferred_element_type=f32)
          + bf_ref[...])
    zs = (jnp.dot(xd, ws[0:D], preferred_element_type=f32)
          + jnp.dot(xs, ws[D:2 * D], preferred_element_type=f32)
          + jnp.dot(ea, ws[2 * D:2 * D + DE], preferred_element_type=f32)
          + bs_ref[...])
    o_ref[...] = _sigmoid(zf) * _softplus(zs)


def _tc_msg(xd, xs, ea, wf, bf, ws, bs):
    E, D = xd.shape
    DE = ea.shape[1]
    BE = 1000
    grid = E // BE
    return pl.pallas_call(
        functools.partial(_msg_body, D, DE),
        grid=(grid,),
        in_specs=[
            pl.BlockSpec((BE, D), lambda i: (i, 0)),
            pl.BlockSpec((BE, D), lambda i: (i, 0)),
            pl.BlockSpec((BE, DE), lambda i: (i, 0)),
            pl.BlockSpec((2 * D + DE, D), lambda i: (0, 0)),
            pl.BlockSpec((2 * D + DE, D), lambda i: (0, 0)),
            pl.BlockSpec((1, D), lambda i: (0, 0)),
            pl.BlockSpec((1, D), lambda i: (0, 0)),
        ],
        out_specs=pl.BlockSpec((BE, D), lambda i: (i, 0)),
        out_shape=jax.ShapeDtypeStruct((E, D), jnp.float32),
    )(xd, xs, ea, wf, ws, bf.reshape(1, D), bs.reshape(1, D))


def _update_body(x_ref, acc_ref, g_ref, b_ref, o_ref):
    N = x_ref.shape[0]
    npad = acc_ref.shape[0] // 2
    y = x_ref[...] + acc_ref[:N, :] + acc_ref[npad:npad + N, :]
    mu = jnp.mean(y, axis=0, keepdims=True)
    var = jnp.mean((y - mu) ** 2, axis=0, keepdims=True)
    yn = g_ref[...] * (y - mu) * lax.rsqrt(var + 1e-5) + b_ref[...]
    o_ref[...] = jnp.maximum(yn, 0.0)


def _tc_update(x, acc, g, b):
    N, D = x.shape
    return pl.pallas_call(
        _update_body,
        out_shape=jax.ShapeDtypeStruct((N, D), jnp.float32),
    )(x, acc, g.reshape(1, D), b.reshape(1, D))


def _bn_rows(y, g, b):
    mu = jnp.mean(y, axis=0, keepdims=True)
    var = jnp.mean((y - mu) ** 2, axis=0, keepdims=True)
    return g * (y - mu) * lax.rsqrt(var + 1e-5) + b


def _head_body(B, D, refs):
    (x_ref, batch_ref, gf_ref, gm_ref, gs_ref,
     sh0w, sh0b, sh0g, sh0bb, sh1w, sh1b, sh1g, sh1bb, *rest) = refs
    ts_refs = rest[:-5]
    outs = rest[-5:]
    x = x_ref[...]
    N = x.shape[0]
    bat = batch_ref[...]
    rows = lax.broadcasted_iota(jnp.int32, (B, N), 0)
    oh = (rows == bat).astype(jnp.float32)
    h = jnp.dot(oh, x, preferred_element_type=jnp.float32)
    gfn = (gf_ref[...] - gm_ref[...]) / (gs_ref[...] + 1e-8)
    w0 = sh0w[...]
    h = (jnp.dot(h, w0[0:D], preferred_element_type=jnp.float32)
         + jnp.dot(gfn, w0[D:], preferred_element_type=jnp.float32)
         + sh0b[...])
    h = jnp.maximum(_bn_rows(h, sh0g[...], sh0bb[...]), 0.0)
    h = jnp.dot(h, sh1w[...], preferred_element_type=jnp.float32) + sh1b[...]
    h = jnp.maximum(_bn_rows(h, sh1g[...], sh1bb[...]), 0.0)
    for t in range(5):
        (w0t, b0t, g0t, bb0t, w1t, b1t, g1t, bb1t, sw, sb) = ts_refs[10 * t:10 * t + 10]
        y = jnp.dot(h, w0t[...], preferred_element_type=jnp.float32) + b0t[...]
        y = jnp.maximum(_bn_rows(y, g0t[...], bb0t[...]), 0.0)
        y = jnp.dot(y, w1t[...], preferred_element_type=jnp.float32) + b1t[...]
        y = jnp.maximum(_bn_rows(y, g1t[...], bb1t[...]), 0.0)
        y = jnp.dot(y, sw[...], preferred_element_type=jnp.float32) + sb[...]
        outs[t][...] = _sigmoid(y)


def _tc_head(x, batch, gf, gm, gs, params):
    N, D = x.shape
    B = gf.shape[0]
    args = [x, batch.reshape(1, N), gf, gm.reshape(1, -1), gs.reshape(1, -1)]
    for i in range(2):
        args += [params[f"sh{i}_W"],
                 params[f"sh{i}_b"].reshape(1, -1),
                 params[f"sh{i}_bng"].reshape(1, -1),
                 params[f"sh{i}_bnb"].reshape(1, -1)]
    for t in range(5):
        for j in range(2):
            args += [params[f"ts{t}_{j}_W"],
                     params[f"ts{t}_{j}_b"].reshape(1, -1),
                     params[f"ts{t}_{j}_bng"].reshape(1, -1),
                     params[f"ts{t}_{j}_bnb"].reshape(1, -1)]
        args += [params[f"ts{t}_sigW"], params[f"ts{t}_sigb"].reshape(1, -1)]

    def body(*refs):
        _head_body(B, D, refs)

    outs = pl.pallas_call(
        body,
        out_shape=[jax.ShapeDtypeStruct((B, 1), jnp.float32)] * 5,
    )(*args)
    return tuple(outs)


def kernel(x, edge_index, edge_attr, batch, n_node_neurons, n_node_features,
           n_edge_neurons, n_edge_features, n_gc_layers, n_s_layers,
           n_ts_layers, use_molecular_descriptors, global_feats, global_mean,
           global_std, params):
    N, D = x.shape
    B = global_feats.shape[0] // global_mean.shape[0]
    src = edge_index[0]
    dst = edge_index[1]
    n_gc = sum(1 for k in params if k.startswith("conv") and k.endswith("_Wf"))
    stacked = tuple(
        jnp.stack([params[f"conv{i}_{nm}"] for i in range(n_gc)])
        for nm in ("Wf", "bf", "Ws", "bs")
    ) + tuple(
        jnp.stack([params[f"bn{i}_{nm}"] for i in range(n_gc)])
        for nm in ("g", "b")
    )

    npad = ((N + NS * 8 - 1) // (NS * 8)) * NS * 8
    zeros = jnp.zeros((npad, D), jnp.float32)

    def layer(xc, w):
        wf, bf, ws, bs, g, b = w
        xd, xs = _sc_gather(xc, dst, src)
        msg = _tc_msg(xd, xs, edge_attr, wf, bf, ws, bs)
        acc = _sc_scatter(msg, dst, zeros, N)
        return _tc_update(xc, acc, g, b), 0

    x, _ = lax.scan(layer, x, stacked)
    gf = global_feats.reshape(B, -1)
    return _tc_head(x, batch, gf, global_mean, global_std, params)


# speedup vs baseline: 4.2871x; 2.0787x over previous
import functools

import jax
import jax.numpy as jnp
from jax import lax
from jax.experimental import pallas as pl
from jax.experimental.pallas import tpu as pltpu
from jax.experimental.pallas import tpu_sc as plsc

NC, NS = 2, 16
KG, NBG = 40, 5
KS, NBS = 40, 5


def _gather_body(nchunks, D, x_hbm, dst_hbm, src_hbm, xds_hbm,
                 idxd, idxs, rowsd, rowss, si, sg, so):
    c = lax.axis_index("c")
    s = lax.axis_index("s")
    wid = c * NS + s
    base = wid * (nchunks * KG)

    def idx_load(i, b):
        pltpu.async_copy(dst_hbm.at[pl.ds(base + i * KG, KG)], idxd.at[b], si.at[b])
        pltpu.async_copy(src_hbm.at[pl.ds(base + i * KG, KG)], idxs.at[b], si.at[b])

    def idx_wait(i, b):
        pltpu.make_async_copy(dst_hbm.at[pl.ds(base + i * KG, KG)], idxd.at[b], si.at[b]).wait()
        pltpu.make_async_copy(src_hbm.at[pl.ds(base + i * KG, KG)], idxs.at[b], si.at[b]).wait()

    def gat_issue(b):
        pltpu.async_copy(x_hbm.at[idxd.at[b]], rowsd.at[b], sg.at[b])
        pltpu.async_copy(x_hbm.at[idxs.at[b]], rowss.at[b], sg.at[b])

    def gat_wait(b):
        pltpu.make_async_copy(x_hbm.at[idxd.at[b]], rowsd.at[b], sg.at[b]).wait()
        pltpu.make_async_copy(x_hbm.at[idxs.at[b]], rowss.at[b], sg.at[b]).wait()

    def out_issue(i, b):
        pltpu.async_copy(rowsd.at[b], xds_hbm.at[pl.ds(base + i * KG, KG), pl.ds(0, D)], so.at[b])
        pltpu.async_copy(rowss.at[b], xds_hbm.at[pl.ds(base + i * KG, KG), pl.ds(D, D)], so.at[b])

    def out_wait(i, b):
        pltpu.make_async_copy(rowsd.at[b], xds_hbm.at[pl.ds(base + i * KG, KG), pl.ds(0, D)], so.at[b]).wait()
        pltpu.make_async_copy(rowss.at[b], xds_hbm.at[pl.ds(base + i * KG, KG), pl.ds(D, D)], so.at[b]).wait()

    for b in range(NBG):
        idx_load(b, b)
    for b in range(NBG):
        idx_wait(b, b)
        gat_issue(b)

    def outer(g, _):
        for b in range(NBG):
            i = g * NBG + b
            gat_wait(b)
            j = i + NBG

            @pl.when(j < nchunks)
            def _pre():
                idx_load(j, b)

            out_issue(i, b)

            @pl.when(j < nchunks)
            def _nxt():
                out_wait(i, b)
                idx_wait(j, b)
                gat_issue(b)
        return 0

    lax.fori_loop(0, nchunks // NBG, outer, 0)
    for b in range(NBG):
        out_wait(nchunks - NBG + b, b)


def _sc_gather(x, dst, src):
    E = dst.shape[0]
    N, D = x.shape
    nchunks = E // (NC * NS * KG)
    mesh = plsc.VectorSubcoreMesh(core_axis_name="c", subcore_axis_name="s")
    f = pl.kernel(
        functools.partial(_gather_body, nchunks, D),
        out_type=jax.ShapeDtypeStruct((E, 2 * D), jnp.float32),
        mesh=mesh,
        scratch_types=[
            pltpu.VMEM((NBG, KG), jnp.int32),
            pltpu.VMEM((NBG, KG), jnp.int32),
            pltpu.VMEM((NBG, KG, D), jnp.float32),
            pltpu.VMEM((NBG, KG, D), jnp.float32),
            pltpu.SemaphoreType.DMA((NBG,)),
            pltpu.SemaphoreType.DMA((NBG,)),
            pltpu.SemaphoreType.DMA((NBG,)),
        ],
    )
    return f(x, dst, src)


def _scatter_body(nchunks, npad, msg_hbm, dst_hbm, zeros_hbm,
                  out_hbm, idx_v, rows_v, si, sm, sa, acc_sh):
    c = lax.axis_index("c")
    s = lax.axis_index("s")
    wid = c * NS + s
    base = wid * (nchunks * KS)

    def ld(i, b):
        pltpu.async_copy(dst_hbm.at[pl.ds(base + i * KS, KS)], idx_v.at[b], si.at[b])
        pltpu.async_copy(msg_hbm.at[pl.ds(base + i * KS, KS)], rows_v.at[b], sm.at[b])

    def ld_wait(i, b):
        pltpu.make_async_copy(dst_hbm.at[pl.ds(base + i * KS, KS)], idx_v.at[b], si.at[b]).wait()
        pltpu.make_async_copy(msg_hbm.at[pl.ds(base + i * KS, KS)], rows_v.at[b], sm.at[b]).wait()

    @pl.when(s == 0)
    def _zero():
        pltpu.sync_copy(zeros_hbm, acc_sh)
    plsc.subcore_barrier()

    for b in range(NBS):
        ld(b, b)

    def outer(g, _):
        for b in range(NBS):
            i = g * NBS + b
            ld_wait(i, b)
            add = pltpu.async_copy(rows_v.at[b], acc_sh.at[idx_v.at[b]], sa.at[b], add=True)
            j = i + NBS

            @pl.when(j < nchunks)
            def _nxt():
                add.wait()
                ld(j, b)
        return 0

    lax.fori_loop(0, nchunks // NBS, outer, 0)
    for b in range(NBS):
        pltpu.make_async_copy(rows_v.at[b], acc_sh.at[idx_v.at[b]], sa.at[b]).wait()
    plsc.subcore_barrier()

    @pl.when(s == 0)
    def _readback():
        pltpu.sync_copy(acc_sh, out_hbm.at[pl.ds(c * npad, npad)])


def _sc_scatter(msg, dst, zeros, N):
    E, D = msg.shape
    nchunks = E // (NC * NS * KS)
    npad = zeros.shape[0]
    mesh = plsc.VectorSubcoreMesh(core_axis_name="c", subcore_axis_name="s")
    f = pl.kernel(
        functools.partial(_scatter_body, nchunks, npad),
        out_type=jax.ShapeDtypeStruct((NC * npad, D), jnp.float32),
        mesh=mesh,
        scratch_types=[
            pltpu.VMEM((NBS, KS), jnp.int32),
            pltpu.VMEM((NBS, KS, D), jnp.float32),
            pltpu.SemaphoreType.DMA((NBS,)),
            pltpu.SemaphoreType.DMA((NBS,)),
            pltpu.SemaphoreType.DMA((NBS,)),
            pltpu.VMEM_SHARED((npad, D), jnp.float32),
        ],
    )
    return f(msg, dst, zeros)


def _sigmoid(z):
    return 1.0 / (1.0 + jnp.exp(-z))


def _softplus(z):
    return jnp.maximum(z, 0.0) + jnp.log(1.0 + jnp.exp(-jnp.abs(z)))


def _msg_body(D, DE, xds_ref, ea_ref, wf_ref, ws_ref, bf_ref, bs_ref,
              o_ref):
    bf16 = jnp.bfloat16
    xds = xds_ref[...].astype(bf16)
    ea = ea_ref[...].astype(bf16)
    wf = wf_ref[...]
    ws = ws_ref[...]
    f32 = jnp.float32
    zf = (jnp.dot(xds, wf[0:2 * D], preferred_element_type=f32)
          + jnp.dot(ea, wf[2 * D:2 * D + DE], preferred_element_type=f32)
          + bf_ref[...])
    zs = (jnp.dot(xds, ws[0:2 * D], preferred_element_type=f32)
          + jnp.dot(ea, ws[2 * D:2 * D + DE], preferred_element_type=f32)
          + bs_ref[...])
    o_ref[...] = _sigmoid(zf) * _softplus(zs)


def _tc_msg(xds, ea, wf, bf, ws, bs):
    E, D2 = xds.shape
    D = D2 // 2
    DE = ea.shape[1]
    BE = 8000
    grid = E // BE
    return pl.pallas_call(
        functools.partial(_msg_body, D, DE),
        grid=(grid,),
        in_specs=[
            pl.BlockSpec((BE, 2 * D), lambda i: (i, 0)),
            pl.BlockSpec((BE, DE), lambda i: (i, 0)),
            pl.BlockSpec((2 * D + DE, D), lambda i: (0, 0)),
            pl.BlockSpec((2 * D + DE, D), lambda i: (0, 0)),
            pl.BlockSpec((1, D), lambda i: (0, 0)),
            pl.BlockSpec((1, D), lambda i: (0, 0)),
        ],
        out_specs=pl.BlockSpec((BE, D), lambda i: (i, 0)),
        out_shape=jax.ShapeDtypeStruct((E, D), jnp.float32),
    )(xds, ea, wf.astype(jnp.bfloat16), ws.astype(jnp.bfloat16),
      bf.reshape(1, D), bs.reshape(1, D))


def _update_body(x_ref, acca_ref, accb_ref, g_ref, b_ref, o_ref):
    N = x_ref.shape[0]
    npad = acca_ref.shape[0] // 2
    y = (x_ref[...] + acca_ref[:N, :] + acca_ref[npad:npad + N, :]
         + accb_ref[:N, :] + accb_ref[npad:npad + N, :])
    mu = jnp.mean(y, axis=0, keepdims=True)
    var = jnp.mean((y - mu) ** 2, axis=0, keepdims=True)
    yn = g_ref[...] * (y - mu) * lax.rsqrt(var + 1e-5) + b_ref[...]
    o_ref[...] = jnp.maximum(yn, 0.0)


def _tc_update(x, acca, accb, g, b):
    N, D = x.shape
    return pl.pallas_call(
        _update_body,
        out_shape=jax.ShapeDtypeStruct((N, D), jnp.float32),
    )(x, acca, accb, g.reshape(1, D), b.reshape(1, D))


def _bn_rows(y, g, b):
    mu = jnp.mean(y, axis=0, keepdims=True)
    var = jnp.mean((y - mu) ** 2, axis=0, keepdims=True)
    return g * (y - mu) * lax.rsqrt(var + 1e-5) + b


def _head_body(B, D, refs):
    (x_ref, batch_ref, gf_ref, gm_ref, gs_ref,
     sh0w, sh0b, sh0g, sh0bb, sh1w, sh1b, sh1g, sh1bb, *rest) = refs
    ts_refs = rest[:-5]
    outs = rest[-5:]
    x = x_ref[...]
    N = x.shape[0]
    bat = batch_ref[...]
    rows = lax.broadcasted_iota(jnp.int32, (B, N), 0)
    oh = (rows == bat).astype(jnp.float32)
    h = jnp.dot(oh, x, preferred_element_type=jnp.float32)
    gfn = (gf_ref[...] - gm_ref[...]) / (gs_ref[...] + 1e-8)
    w0 = sh0w[...]
    h = (jnp.dot(h, w0[0:D], preferred_element_type=jnp.float32)
         + jnp.dot(gfn, w0[D:], preferred_element_type=jnp.float32)
         + sh0b[...])
    h = jnp.maximum(_bn_rows(h, sh0g[...], sh0bb[...]), 0.0)
    h = jnp.dot(h, sh1w[...], preferred_element_type=jnp.float32) + sh1b[...]
    h = jnp.maximum(_bn_rows(h, sh1g[...], sh1bb[...]), 0.0)
    for t in range(5):
        (w0t, b0t, g0t, bb0t, w1t, b1t, g1t, bb1t, sw, sb) = ts_refs[10 * t:10 * t + 10]
        y = jnp.dot(h, w0t[...], preferred_element_type=jnp.float32) + b0t[...]
        y = jnp.maximum(_bn_rows(y, g0t[...], bb0t[...]), 0.0)
        y = jnp.dot(y, w1t[...], preferred_element_type=jnp.float32) + b1t[...]
        y = jnp.maximum(_bn_rows(y, g1t[...], bb1t[...]), 0.0)
        y = jnp.dot(y, sw[...], preferred_element_type=jnp.float32) + sb[...]
        outs[t][...] = _sigmoid(y)


def _tc_head(x, batch, gf, gm, gs, params):
    N, D = x.shape
    B = gf.shape[0]
    args = [x, batch.reshape(1, N), gf, gm.reshape(1, -1), gs.reshape(1, -1)]
    for i in range(2):
        args += [params[f"sh{i}_W"],
                 params[f"sh{i}_b"].reshape(1, -1),
                 params[f"sh{i}_bng"].reshape(1, -1),
                 params[f"sh{i}_bnb"].reshape(1, -1)]
    for t in range(5):
        for j in range(2):
            args += [params[f"ts{t}_{j}_W"],
                     params[f"ts{t}_{j}_b"].reshape(1, -1),
                     params[f"ts{t}_{j}_bng"].reshape(1, -1),
                     params[f"ts{t}_{j}_bnb"].reshape(1, -1)]
        args += [params[f"ts{t}_sigW"], params[f"ts{t}_sigb"].reshape(1, -1)]

    def body(*refs):
        _head_body(B, D, refs)

    outs = pl.pallas_call(
        body,
        out_shape=[jax.ShapeDtypeStruct((B, 1), jnp.float32)] * 5,
    )(*args)
    return tuple(outs)


def kernel(x, edge_index, edge_attr, batch, n_node_neurons, n_node_features,
           n_edge_neurons, n_edge_features, n_gc_layers, n_s_layers,
           n_ts_layers, use_molecular_descriptors, global_feats, global_mean,
           global_std, params):
    N, D = x.shape
    B = global_feats.shape[0] // global_mean.shape[0]
    src = edge_index[0]
    dst = edge_index[1]
    n_gc = sum(1 for k in params if k.startswith("conv") and k.endswith("_Wf"))
    npad = ((N + NS * 8 - 1) // (NS * 8)) * NS * 8
    zeros = jnp.zeros((npad, D), jnp.float32)
    E = dst.shape[0]
    Eh = E // 2
    dstA, dstB = dst[:Eh], dst[Eh:]
    srcA, srcB = src[:Eh], src[Eh:]
    eaA, eaB = edge_attr[:Eh], edge_attr[Eh:]

    for i in range(n_gc):
        wf, bf = params[f"conv{i}_Wf"], params[f"conv{i}_bf"]
        ws, bs = params[f"conv{i}_Ws"], params[f"conv{i}_bs"]
        xdsA = _sc_gather(x, dstA, srcA)
        xdsB = _sc_gather(x, dstB, srcB)
        msgA = _tc_msg(xdsA, eaA, wf, bf, ws, bs)
        msgB = _tc_msg(xdsB, eaB, wf, bf, ws, bs)
        accA = _sc_scatter(msgA, dstA, zeros, N)
        accB = _sc_scatter(msgB, dstB, zeros, N)
        x = _tc_update(x, accA, accB, params[f"bn{i}_g"], params[f"bn{i}_b"])
    gf = global_feats.reshape(B, -1)
    return _tc_head(x, batch, gf, global_mean, global_std, params)
